# baseline (speedup 1.0000x reference)
import functools

import jax
import jax.numpy as jnp
from jax import lax
from jax.experimental import pallas as pl
from jax.experimental.pallas import tpu as pltpu
from jax.experimental.pallas import tpu_sc as plsc

NTOK = 2048
ACT_DIM = 768
DICT = 24576
K = 64

BLK = 256
NBLK = DICT // BLK
CAPL = 32
NCAP = 16 * CAPL

BM_ = 512
BN_ = 2048


def _enc_body(x_ref, w_ref, benc_ref, bdec_ref, out_ref, bmt_ref):
    xm = x_ref[...] - bdec_ref[...][None, :]
    acc = lax.dot_general(xm, w_ref[...], (((1,), (1,)), ((), ())),
                          preferred_element_type=jnp.float32)
    p = jnp.maximum(acc + benc_ref[...][None, :], 0.0)
    out_ref[...] = p
    for c in range(BN_ // BLK):
        bmt_ref[c, :] = jnp.max(p[:, c * BLK:(c + 1) * BLK], axis=1)


def _encode(x, W_enc, b_enc, b_dec):
    grid = (NTOK // BM_, DICT // BN_)
    return pl.pallas_call(
        _enc_body,
        grid=grid,
        in_specs=[
            pl.BlockSpec((BM_, ACT_DIM), lambda i, j: (i, 0)),
            pl.BlockSpec((BN_, ACT_DIM), lambda i, j: (j, 0)),
            pl.BlockSpec((BN_,), lambda i, j: (j,)),
            pl.BlockSpec((ACT_DIM,), lambda i, j: (0,)),
        ],
        out_specs=[
            pl.BlockSpec((BM_, BN_), lambda i, j: (i, j)),
            pl.BlockSpec((BN_ // BLK, BM_), lambda i, j: (j, i)),
        ],
        out_shape=[
            jax.ShapeDtypeStruct((NTOK, DICT), jnp.float32),
            jax.ShapeDtypeStruct((NBLK, NTOK), jnp.float32),
        ],
    )(x, W_enc, b_enc, b_dec)


def _make_compress():
    info = plsc.get_sparse_core_info()
    NC, NS, L = info.num_cores, info.num_subcores, info.num_lanes
    NW = NC * NS
    t_per_w = NTOK // NW
    NV = DICT // L

    mesh = plsc.VectorSubcoreMesh(core_axis_name="c", subcore_axis_name="s")

    @functools.partial(
        pl.kernel,
        mesh=mesh,
        compiler_params=pltpu.CompilerParams(needs_layout_passes=False),
        out_type=[
            jax.ShapeDtypeStruct((NTOK, NCAP), jnp.float32),
            jax.ShapeDtypeStruct((NTOK, NCAP), jnp.int32),
        ],
        scratch_types=[
            pltpu.VMEM((DICT,), jnp.float32),
            pltpu.VMEM((DICT,), jnp.float32),
            pltpu.VMEM((t_per_w, NBLK), jnp.float32),
            pltpu.VMEM((NCAP,), jnp.float32),
            pltpu.VMEM((NCAP,), jnp.int32),
            pltpu.SemaphoreType.DMA,
            pltpu.SemaphoreType.DMA,
        ],
    )
    def compress(p_hbm, bm_hbm, sval_hbm, sidx_hbm,
                 rowa_v, rowb_v, bmall_v, sv_v, si_v, sema, semb):
        cid = lax.axis_index("c")
        sid = lax.axis_index("s")
        wid = sid * NC + cid
        t0 = wid * t_per_w
        lane = lax.iota(jnp.int32, L)
        neginf = jnp.full((L,), -jnp.inf, jnp.float32)

        pltpu.sync_copy(bm_hbm.at[pl.ds(t0, t_per_w)], bmall_v)

        def zinit(c, _):
            si_v[pl.ds(c * L, L)] = jnp.zeros((L,), jnp.int32)
            return 0

        lax.fori_loop(0, CAPL, zinit, 0)
        pltpu.async_copy(p_hbm.at[t0], rowa_v, sema).wait()

        def do_row(t, row_v, ti):
            bmu = [lax.bitcast_convert_type(bmall_v[ti, pl.ds(c * L, L)], jnp.int32)
                   for c in range(NBLK // L)]
            tau = jnp.zeros((L,), jnp.int32)
            for bit in range(30, -1, -1):
                t2 = tau | (1 << bit)
                cnt = jnp.zeros((L,), jnp.int32)
                for u in bmu:
                    cnt = cnt + plsc.all_reduce_population_count(u >= t2)
                tau = jnp.where(cnt >= K, t2, tau)
            tauf = lax.bitcast_convert_type(tau, jnp.float32)


            def batch(b, pos):
                base = b * L * 4
                for s in range(4):
                    v = row_v[pl.ds(base + s * L, L)]
                    m = v >= tauf
                    mm = m & (pos < CAPL)
                    tgt = pos * L + lane
                    plsc.store_scatter(
                        si_v, [tgt],
                        jnp.full((L,), base + s * L, jnp.int32) + lane,
                        mask=mm)
                    pos = pos + jnp.where(m, 1, 0)
                return pos

            pos = lax.fori_loop(0, NV // 4, batch, jnp.zeros((L,), jnp.int32))

            def post(c, _):
                idxv = si_v[pl.ds(c * L, L)]
                g = plsc.load_gather(row_v, [idxv])
                sv_v[pl.ds(c * L, L)] = jnp.where(
                    jnp.full((L,), 1, jnp.int32) * c < pos, g, neginf)
                return 0

            lax.fori_loop(0, CAPL, post, 0)
            pltpu.sync_copy(sv_v, sval_hbm.at[t])
            pltpu.sync_copy(si_v, sidx_hbm.at[t])

        def pair(i, _):
            ta = t0 + 2 * i
            pltpu.async_copy(p_hbm.at[ta + 1], rowb_v, semb)
            do_row(ta, rowa_v, 2 * i)

            @pl.when(i < t_per_w // 2 - 1)
            def _():
                pltpu.async_copy(p_hbm.at[ta + 2], rowa_v, sema)

            pltpu.make_async_copy(p_hbm.at[ta + 1], rowb_v, semb).wait()
            do_row(ta + 1, rowb_v, 2 * i + 1)

            @pl.when(i < t_per_w // 2 - 1)
            def _():
                pltpu.make_async_copy(p_hbm.at[ta + 2], rowa_v, sema).wait()
            return 0

        lax.fori_loop(0, t_per_w // 2, pair, 0)

    return compress


TB = 256


def _sel_body(sv_ref, si_ref, vals_ref, idx_ref, buf_ref):
    buf_ref[...] = sv_ref[...]
    sidx = si_ref[...]
    kiota = lax.broadcasted_iota(jnp.int32, (TB, K), 1)
    BIG = jnp.int32(2**30)

    def step(j, carry):
        vacc, iacc = carry
        buf = buf_ref[...]
        m = jnp.max(buf, axis=1, keepdims=True)
        hit = buf == m
        ix = jnp.min(jnp.where(hit, sidx, BIG), axis=1, keepdims=True)
        buf_ref[...] = jnp.where(hit & (sidx == ix), -jnp.inf, buf)
        vacc = jnp.where(kiota == j, m, vacc)
        iacc = jnp.where(kiota == j, ix, iacc)
        return vacc, iacc

    vacc = jnp.zeros((TB, K), jnp.float32)
    iacc = jnp.zeros((TB, K), jnp.int32)
    vacc, iacc = lax.fori_loop(0, K, step, (vacc, iacc))
    vals_ref[...] = vacc
    idx_ref[...] = iacc


def _select(svals, sidx):
    return pl.pallas_call(
        _sel_body,
        grid=(NTOK // TB,),
        in_specs=[
            pl.BlockSpec((TB, NCAP), lambda i: (i, 0)),
            pl.BlockSpec((TB, NCAP), lambda i: (i, 0)),
        ],
        out_specs=[
            pl.BlockSpec((TB, K), lambda i: (i, 0)),
            pl.BlockSpec((TB, K), lambda i: (i, 0)),
        ],
        out_shape=[
            jax.ShapeDtypeStruct((NTOK, K), jnp.float32),
            jax.ShapeDtypeStruct((NTOK, K), jnp.int32),
        ],
        scratch_shapes=[pltpu.VMEM((TB, NCAP), jnp.float32)],
    )(svals, sidx)


def _make_decode():
    info = plsc.get_sparse_core_info()
    NC, NS, L = info.num_cores, info.num_subcores, info.num_lanes
    NW = NC * NS
    t_per_w = NTOK // NW
    G = ACT_DIM // L

    mesh = plsc.VectorSubcoreMesh(core_axis_name="c", subcore_axis_name="s")

    @functools.partial(
        pl.kernel,
        mesh=mesh,
        compiler_params=pltpu.CompilerParams(needs_layout_passes=False),
        out_type=jax.ShapeDtypeStruct((NTOK, ACT_DIM), jnp.float32),
        scratch_types=[
            pltpu.VMEM((t_per_w, K), jnp.int32),
            pltpu.VMEM((t_per_w, K), jnp.float32),
            pltpu.VMEM((K, ACT_DIM), jnp.float32),
            pltpu.VMEM((K, ACT_DIM), jnp.float32),
            pltpu.VMEM((ACT_DIM,), jnp.float32),
            pltpu.VMEM((ACT_DIM,), jnp.float32),
            pltpu.SemaphoreType.DMA,
            pltpu.SemaphoreType.DMA,
        ],
    )
    def decode(wdt_hbm, bdec_hbm, vals_hbm, idx_hbm, out_hbm,
               idx_v, vals_v, rowsa_v, rowsb_v, acc_v, bdec_v, sema, semb):
        cid = lax.axis_index("c")
        sid = lax.axis_index("s")
        wid = sid * NC + cid
        t0 = wid * t_per_w
        pltpu.sync_copy(bdec_hbm, bdec_v)
        pltpu.sync_copy(idx_hbm.at[pl.ds(t0, t_per_w)], idx_v)
        pltpu.sync_copy(vals_hbm.at[pl.ds(t0, t_per_w)], vals_v)
        pltpu.async_copy(wdt_hbm.at[idx_v.at[0]], rowsa_v, sema).wait()

        def do_token(t, ti, rows_v):
            sps = []
            for kb in range(K // L):
                vv = vals_v[ti, pl.ds(kb * L, L)]
                sps.extend(vv[j] for j in range(L))

            def gstep(g, _):
                base = g * L
                a = bdec_v[pl.ds(base, L)]
                for k in range(K):
                    a = a + sps[k] * rows_v[k, pl.ds(base, L)]
                acc_v[pl.ds(base, L)] = a
                return 0

            lax.fori_loop(0, G, gstep, 0)
            pltpu.sync_copy(acc_v, out_hbm.at[t])

        def pair(i, _):
            ta = t0 + 2 * i
            pltpu.async_copy(wdt_hbm.at[idx_v.at[2 * i + 1]], rowsb_v, semb)
            do_token(ta, 2 * i, rowsa_v)

            @pl.when(i < t_per_w // 2 - 1)
            def _():
                pltpu.async_copy(wdt_hbm.at[idx_v.at[2 * i + 2]], rowsa_v, sema)

            pltpu.make_async_copy(wdt_hbm.at[idx_v.at[0]], rowsb_v, semb).wait()
            do_token(ta + 1, 2 * i + 1, rowsb_v)

            @pl.when(i < t_per_w // 2 - 1)
            def _():
                pltpu.make_async_copy(wdt_hbm.at[idx_v.at[0]], rowsa_v, sema).wait()
            return 0

        lax.fori_loop(0, t_per_w // 2, pair, 0)

    return decode


def kernel(x, W_enc, b_enc, W_dec, b_dec):
    pre_acts, bmt = _encode(x, W_enc, b_enc, b_dec)
    bm = bmt.T
    svals, sidx = _make_compress()(pre_acts, bm)
    top_vals, top_idx = _select(svals, sidx)
    wdt = W_dec.T
    recon = _make_decode()(wdt, b_dec, top_vals, top_idx)
    return recon, top_vals, top_idx


# speedup vs baseline: 1.0845x; 1.0845x over previous
import functools

import jax
import jax.numpy as jnp
from jax import lax
from jax.experimental import pallas as pl
from jax.experimental.pallas import tpu as pltpu
from jax.experimental.pallas import tpu_sc as plsc

NTOK = 2048
ACT_DIM = 768
DICT = 24576
K = 64

BLK = 256
NBLK = DICT // BLK
CAPL = 32
NCAP = 16 * CAPL

BM_ = 512
BN_ = 2048


def _enc_body(x_ref, w_ref, benc_ref, bdec_ref, out_ref, bmt_ref):
    xm = x_ref[...] - bdec_ref[...][None, :]
    acc = lax.dot_general(xm, w_ref[...], (((1,), (1,)), ((), ())),
                          preferred_element_type=jnp.float32)
    p = jnp.maximum(acc + benc_ref[...][None, :], 0.0)
    out_ref[...] = p
    for c in range(BN_ // BLK):
        bmt_ref[c, :] = jnp.max(p[:, c * BLK:(c + 1) * BLK], axis=1)


def _encode(x, W_enc, b_enc, b_dec):
    grid = (NTOK // BM_, DICT // BN_)
    return pl.pallas_call(
        _enc_body,
        grid=grid,
        in_specs=[
            pl.BlockSpec((BM_, ACT_DIM), lambda i, j: (i, 0)),
            pl.BlockSpec((BN_, ACT_DIM), lambda i, j: (j, 0)),
            pl.BlockSpec((BN_,), lambda i, j: (j,)),
            pl.BlockSpec((ACT_DIM,), lambda i, j: (0,)),
        ],
        out_specs=[
            pl.BlockSpec((BM_, BN_), lambda i, j: (i, j)),
            pl.BlockSpec((BN_ // BLK, BM_), lambda i, j: (j, i)),
        ],
        out_shape=[
            jax.ShapeDtypeStruct((NTOK, DICT), jnp.float32),
            jax.ShapeDtypeStruct((NBLK, NTOK), jnp.float32),
        ],
    )(x, W_enc, b_enc, b_dec)


def _make_compress():
    info = plsc.get_sparse_core_info()
    NC, NS, L = info.num_cores, info.num_subcores, info.num_lanes
    NW = NC * NS
    t_per_w = NTOK // NW
    NV = DICT // L

    mesh = plsc.VectorSubcoreMesh(core_axis_name="c", subcore_axis_name="s")

    @functools.partial(
        pl.kernel,
        mesh=mesh,
        compiler_params=pltpu.CompilerParams(needs_layout_passes=False),
        out_type=[
            jax.ShapeDtypeStruct((NTOK, NCAP), jnp.float32),
            jax.ShapeDtypeStruct((NTOK, NCAP), jnp.int32),
        ],
        scratch_types=[
            pltpu.VMEM((DICT,), jnp.float32),
            pltpu.VMEM((DICT,), jnp.float32),
            pltpu.VMEM((t_per_w, NBLK), jnp.float32),
            pltpu.VMEM((NCAP,), jnp.float32),
            pltpu.VMEM((NCAP,), jnp.int32),
            pltpu.SemaphoreType.DMA,
            pltpu.SemaphoreType.DMA,
        ],
    )
    def compress(p_hbm, bm_hbm, sval_hbm, sidx_hbm,
                 rowa_v, rowb_v, bmall_v, sv_v, si_v, sema, semb):
        cid = lax.axis_index("c")
        sid = lax.axis_index("s")
        wid = sid * NC + cid
        t0 = wid * t_per_w
        lane = lax.iota(jnp.int32, L)
        neginf = jnp.full((L,), -jnp.inf, jnp.float32)

        pltpu.sync_copy(bm_hbm.at[pl.ds(t0, t_per_w)], bmall_v)
        pltpu.async_copy(p_hbm.at[t0], rowa_v, sema).wait()

        def do_row(t, row_v, ti):
            bmu = [lax.bitcast_convert_type(bmall_v[ti, pl.ds(c * L, L)], jnp.int32)
                   for c in range(NBLK // L)]
            tau = jnp.zeros((L,), jnp.int32)
            for bit in range(30, -1, -1):
                t2 = tau | (1 << bit)
                cnt = jnp.zeros((L,), jnp.int32)
                for u in bmu:
                    cnt = cnt + plsc.all_reduce_population_count(u >= t2)
                tau = jnp.where(cnt >= K, t2, tau)
            tauf = lax.bitcast_convert_type(tau, jnp.float32)


            for s in range(CAPL):
                sv_v[pl.ds(s * L, L)] = neginf

            def batch(b, pos):
                base = b * L * 4
                vs, ms = [], []
                anym = None
                for s in range(4):
                    v = row_v[pl.ds(base + s * L, L)]
                    m = v >= tauf
                    vs.append(v)
                    ms.append(m)
                    anym = m if anym is None else (anym | m)
                cnt = plsc.all_reduce_population_count(anym)[0]

                @pl.when(cnt > 0)
                def _():
                    p2 = pos
                    for s in range(4):
                        mm = ms[s] & (p2 < CAPL)
                        tgt = p2 * L + lane
                        plsc.store_scatter(sv_v, [tgt], vs[s], mask=mm)
                        plsc.store_scatter(
                            si_v, [tgt],
                            jnp.full((L,), base + s * L, jnp.int32) + lane,
                            mask=mm)
                        p2 = p2 + jnp.where(ms[s], 1, 0)

                for s in range(4):
                    pos = pos + jnp.where(ms[s], 1, 0)
                return pos

            lax.fori_loop(0, NV // 4, batch, jnp.zeros((L,), jnp.int32))
            pltpu.sync_copy(sv_v, sval_hbm.at[t])
            pltpu.sync_copy(si_v, sidx_hbm.at[t])

        def pair(i, _):
            ta = t0 + 2 * i
            pltpu.async_copy(p_hbm.at[ta + 1], rowb_v, semb)
            do_row(ta, rowa_v, 2 * i)

            @pl.when(i < t_per_w // 2 - 1)
            def _():
                pltpu.async_copy(p_hbm.at[ta + 2], rowa_v, sema)

            pltpu.make_async_copy(p_hbm.at[ta + 1], rowb_v, semb).wait()
            do_row(ta + 1, rowb_v, 2 * i + 1)

            @pl.when(i < t_per_w // 2 - 1)
            def _():
                pltpu.make_async_copy(p_hbm.at[ta + 2], rowa_v, sema).wait()
            return 0

        lax.fori_loop(0, t_per_w // 2, pair, 0)

    return compress


TB = 256


def _sel_body(sv_ref, si_ref, vals_ref, idx_ref, buf_ref):
    buf_ref[...] = sv_ref[...]
    sidx = si_ref[...]
    kiota = lax.broadcasted_iota(jnp.int32, (TB, K), 1)
    BIG = jnp.int32(2**30)

    def step(j, carry):
        vacc, iacc = carry
        buf = buf_ref[...]
        m = jnp.max(buf, axis=1, keepdims=True)
        hit = buf == m
        ix = jnp.min(jnp.where(hit, sidx, BIG), axis=1, keepdims=True)
        buf_ref[...] = jnp.where(hit & (sidx == ix), -jnp.inf, buf)
        vacc = jnp.where(kiota == j, m, vacc)
        iacc = jnp.where(kiota == j, ix, iacc)
        return vacc, iacc

    vacc = jnp.zeros((TB, K), jnp.float32)
    iacc = jnp.zeros((TB, K), jnp.int32)
    vacc, iacc = lax.fori_loop(0, K, step, (vacc, iacc))
    vals_ref[...] = vacc
    idx_ref[...] = iacc


def _select(svals, sidx):
    return pl.pallas_call(
        _sel_body,
        grid=(NTOK // TB,),
        in_specs=[
            pl.BlockSpec((TB, NCAP), lambda i: (i, 0)),
            pl.BlockSpec((TB, NCAP), lambda i: (i, 0)),
        ],
        out_specs=[
            pl.BlockSpec((TB, K), lambda i: (i, 0)),
            pl.BlockSpec((TB, K), lambda i: (i, 0)),
        ],
        out_shape=[
            jax.ShapeDtypeStruct((NTOK, K), jnp.float32),
            jax.ShapeDtypeStruct((NTOK, K), jnp.int32),
        ],
        scratch_shapes=[pltpu.VMEM((TB, NCAP), jnp.float32)],
    )(svals, sidx)


def _make_decode():
    info = plsc.get_sparse_core_info()
    NC, NS, L = info.num_cores, info.num_subcores, info.num_lanes
    NW = NC * NS
    t_per_w = NTOK // NW
    G = ACT_DIM // L

    mesh = plsc.VectorSubcoreMesh(core_axis_name="c", subcore_axis_name="s")

    @functools.partial(
        pl.kernel,
        mesh=mesh,
        compiler_params=pltpu.CompilerParams(needs_layout_passes=False),
        out_type=jax.ShapeDtypeStruct((NTOK, ACT_DIM), jnp.float32),
        scratch_types=[
            pltpu.VMEM((t_per_w, K), jnp.int32),
            pltpu.VMEM((t_per_w, K), jnp.float32),
            pltpu.VMEM((K, ACT_DIM), jnp.float32),
            pltpu.VMEM((K, ACT_DIM), jnp.float32),
            pltpu.VMEM((ACT_DIM,), jnp.float32),
            pltpu.VMEM((ACT_DIM,), jnp.float32),
            pltpu.SemaphoreType.DMA,
            pltpu.SemaphoreType.DMA,
        ],
    )
    def decode(wdt_hbm, bdec_hbm, vals_hbm, idx_hbm, out_hbm,
               idx_v, vals_v, rowsa_v, rowsb_v, acc_v, bdec_v, sema, semb):
        cid = lax.axis_index("c")
        sid = lax.axis_index("s")
        wid = sid * NC + cid
        t0 = wid * t_per_w
        pltpu.sync_copy(bdec_hbm, bdec_v)
        pltpu.sync_copy(idx_hbm.at[pl.ds(t0, t_per_w)], idx_v)
        pltpu.sync_copy(vals_hbm.at[pl.ds(t0, t_per_w)], vals_v)
        pltpu.async_copy(wdt_hbm.at[idx_v.at[0]], rowsa_v, sema).wait()

        def do_token(t, ti, rows_v):
            sps = []
            for kb in range(K // L):
                vv = vals_v[ti, pl.ds(kb * L, L)]
                sps.extend(vv[j] for j in range(L))

            def gstep(g, _):
                base = g * L
                accs = [bdec_v[pl.ds(base, L)], 0.0, 0.0, 0.0]
                for k in range(K):
                    accs[k % 4] = accs[k % 4] + sps[k] * rows_v[k, pl.ds(base, L)]
                acc_v[pl.ds(base, L)] = (accs[0] + accs[1]) + (accs[2] + accs[3])
                return 0

            lax.fori_loop(0, G, gstep, 0)
            pltpu.sync_copy(acc_v, out_hbm.at[t])

        def pair(i, _):
            ta = t0 + 2 * i
            pltpu.async_copy(wdt_hbm.at[idx_v.at[2 * i + 1]], rowsb_v, semb)
            do_token(ta, 2 * i, rowsa_v)

            @pl.when(i < t_per_w // 2 - 1)
            def _():
                pltpu.async_copy(wdt_hbm.at[idx_v.at[2 * i + 2]], rowsa_v, sema)

            pltpu.make_async_copy(wdt_hbm.at[idx_v.at[0]], rowsb_v, semb).wait()
            do_token(ta + 1, 2 * i + 1, rowsb_v)

            @pl.when(i < t_per_w // 2 - 1)
            def _():
                pltpu.make_async_copy(wdt_hbm.at[idx_v.at[0]], rowsa_v, sema).wait()
            return 0

        lax.fori_loop(0, t_per_w // 2, pair, 0)

    return decode


def kernel(x, W_enc, b_enc, W_dec, b_dec):
    pre_acts, bmt = _encode(x, W_enc, b_enc, b_dec)
    bm = bmt.T
    svals, sidx = _make_compress()(pre_acts, bm)
    top_vals, top_idx = _select(svals, sidx)
    wdt = W_dec.T
    recon = _make_decode()(wdt, b_dec, top_vals, top_idx)
    return recon, top_vals, top_idx
